# Initial kernel scaffold; baseline (speedup 1.0000x reference)
import jax
import jax.numpy as jnp
from jax.experimental import pallas as pl


def kernel(x, edge_index, hop_edge_index, hop_edge_att, W1, b1, W2, b2):
    raise NotImplementedError("write your pallas kernel here")


# speedup vs baseline: 4.4275x; 4.4275x over previous
import functools

import jax
import jax.numpy as jnp
from jax import lax
from jax.experimental import pallas as pl
from jax.experimental.pallas import tpu as pltpu
from jax.experimental.pallas import tpu_sc as plsc

N = 10000
D = 128
H = 256
C = 40
CP = 48
NPAD = 10240
NC, NS, LANES = 2, 16, 16
NW = NC * NS
CHUNK = 128
RBLK = 400


def _mlp_body(x_ref, w1_ref, b1_ref, w2_ref, b2_ref, h_ref):
    h1 = jnp.dot(x_ref[...], w1_ref[...], preferred_element_type=jnp.float32)
    h1 = jnp.maximum(h1 + b1_ref[...], 0.0)
    h_ref[...] = jnp.dot(h1, w2_ref[...],
                         preferred_element_type=jnp.float32) + b2_ref[...]


def _combine_body(h_ref, a0_ref, a1_ref, o_ref):
    s = h_ref[...] + a0_ref[...] + a1_ref[...]
    col = lax.broadcasted_iota(jnp.int32, (RBLK, CP), 1)
    valid = col < C
    masked = jnp.where(valid, s, -jnp.inf)
    m = jnp.max(masked, axis=1, keepdims=True)
    ex = jnp.where(valid, jnp.exp(s - m), 0.0)
    lse = jnp.log(jnp.sum(ex, axis=1, keepdims=True)) + m
    o_ref[...] = s - lse


def _make_prop(cpt):
    slab = NPAD // NS

    def body(h_hbm, src_hbm, dst_hbm, att_hbm, out_hbm,
             src_v, dst_v, att_v, rows_v, acc_sh):
        cid = lax.axis_index("c")
        sid = lax.axis_index("s")
        wid = sid * NC + cid
        row0 = wid * cpt

        pltpu.sync_copy(src_hbm.at[pl.ds(row0, cpt)], src_v)
        pltpu.sync_copy(dst_hbm.at[pl.ds(row0, cpt)], dst_v)
        pltpu.sync_copy(att_hbm.at[pl.ds(row0, cpt)], att_v)

        zeros16 = jnp.zeros((LANES,), jnp.float32)

        @pl.loop(0, CHUNK)
        def _zero(e):
            for cc in range(CP // LANES):
                rows_v[e, pl.ds(cc * LANES, LANES)] = zeros16

        slab0 = sid * slab
        for i in range(slab // CHUNK):
            pltpu.sync_copy(rows_v, acc_sh.at[pl.ds(slab0 + i * CHUNK, CHUNK)])
        plsc.subcore_barrier()

        @pl.loop(0, cpt)
        def _edges(j):
            pltpu.sync_copy(h_hbm.at[src_v.at[j]], rows_v)

            @pl.loop(0, CHUNK // LANES)
            def _scale(e16):
                av = att_v[j, pl.ds(e16 * LANES, LANES)]
                for l in range(LANES):
                    a = av[l]
                    e = e16 * LANES + l
                    for cc in range(CP // LANES):
                        sl = pl.ds(cc * LANES, LANES)
                        rows_v[e, sl] = rows_v[e, sl] * a

            pltpu.sync_copy(rows_v, acc_sh.at[dst_v.at[j]], add=True)

        plsc.subcore_barrier()
        pltpu.sync_copy(acc_sh.at[pl.ds(slab0, slab)],
                        out_hbm.at[cid, pl.ds(slab0, slab)])

    mesh = plsc.VectorSubcoreMesh(core_axis_name="c", subcore_axis_name="s")
    return pl.kernel(
        body,
        out_type=jax.ShapeDtypeStruct((NC, NPAD, CP), jnp.float32),
        mesh=mesh,
        compiler_params=pltpu.CompilerParams(use_tc_tiling_on_sc=False),
        scratch_types=[
            pltpu.VMEM((cpt, CHUNK), jnp.int32),
            pltpu.VMEM((cpt, CHUNK), jnp.int32),
            pltpu.VMEM((cpt, CHUNK), jnp.float32),
            pltpu.VMEM((CHUNK, CP), jnp.float32),
            pltpu.VMEM_SHARED((NPAD, CP), jnp.float32),
        ],
    )


def kernel(x, edge_index, hop_edge_index, hop_edge_att, W1, b1, W2, b2):
    w2p = jnp.pad(W2, ((0, 0), (0, CP - C)))
    b2p = jnp.pad(b2, (0, CP - C)).reshape(1, CP)
    b1r = b1.reshape(1, H)
    h = pl.pallas_call(
        _mlp_body,
        grid=(N // RBLK,),
        in_specs=[
            pl.BlockSpec((RBLK, D), lambda i: (i, 0)),
            pl.BlockSpec((D, H), lambda i: (0, 0)),
            pl.BlockSpec((1, H), lambda i: (0, 0)),
            pl.BlockSpec((H, CP), lambda i: (0, 0)),
            pl.BlockSpec((1, CP), lambda i: (0, 0)),
        ],
        out_specs=pl.BlockSpec((RBLK, CP), lambda i: (i, 0)),
        out_shape=jax.ShapeDtypeStruct((N, CP), jnp.float32),
    )(x, W1, b1r, w2p, b2p)

    src = hop_edge_index[:, 0, :].reshape(-1)
    dst = hop_edge_index[:, 1, :].reshape(-1)
    att = hop_edge_att.reshape(-1)
    etot = att.shape[0]
    cpt = -(-etot // (NW * CHUNK))
    cpt = -(-cpt // 8) * 8
    epad = NW * cpt * CHUNK
    pad = epad - etot
    src = jnp.concatenate([src, jnp.zeros((pad,), src.dtype)]).reshape(-1, CHUNK)
    dst = jnp.concatenate([dst, jnp.zeros((pad,), dst.dtype)]).reshape(-1, CHUNK)
    att = jnp.concatenate([att, jnp.zeros((pad,), att.dtype)]).reshape(-1, CHUNK)

    acc = _make_prop(cpt)(h, src, dst, att)

    out = pl.pallas_call(
        _combine_body,
        grid=(N // RBLK,),
        in_specs=[
            pl.BlockSpec((RBLK, CP), lambda i: (i, 0)),
            pl.BlockSpec((RBLK, CP), lambda i: (i, 0)),
            pl.BlockSpec((RBLK, CP), lambda i: (i, 0)),
        ],
        out_specs=pl.BlockSpec((RBLK, CP), lambda i: (i, 0)),
        out_shape=jax.ShapeDtypeStruct((N, CP), jnp.float32),
    )(h, acc[0, :N], acc[1, :N])
    return out[:, :C]


# speedup vs baseline: 5.6540x; 1.2770x over previous
import functools

import jax
import jax.numpy as jnp
from jax import lax
from jax.experimental import pallas as pl
from jax.experimental.pallas import tpu as pltpu
from jax.experimental.pallas import tpu_sc as plsc

N = 10000
D = 128
H = 256
C = 40
CP = 48
NPAD = 10240
NC, NS, LANES = 2, 16, 16
NW = NC * NS
CHUNK = 128
RBLK = 400


def _mlp_body(x_ref, w1_ref, b1_ref, w2_ref, b2_ref, h_ref):
    h1 = jnp.dot(x_ref[...], w1_ref[...], preferred_element_type=jnp.float32)
    h1 = jnp.maximum(h1 + b1_ref[...], 0.0)
    h_ref[...] = jnp.dot(h1, w2_ref[...],
                         preferred_element_type=jnp.float32) + b2_ref[...]


def _combine_body(h_ref, a0_ref, a1_ref, o_ref):
    s = h_ref[...] + a0_ref[...] + a1_ref[...]
    col = lax.broadcasted_iota(jnp.int32, (RBLK, CP), 1)
    valid = col < C
    masked = jnp.where(valid, s, -jnp.inf)
    m = jnp.max(masked, axis=1, keepdims=True)
    ex = jnp.where(valid, jnp.exp(s - m), 0.0)
    lse = jnp.log(jnp.sum(ex, axis=1, keepdims=True)) + m
    o_ref[...] = s - lse


BLK = 80


def _make_prop(cpt):
    slab = NPAD // NS
    nblk = cpt // BLK

    def body(h_hbm, src_hbm, dst_hbm, att_hbm, out_hbm,
             src_v, dst_v, att_v, rows0, rows1, acc_sh, sem0, sem1):
        cid = lax.axis_index("c")
        sid = lax.axis_index("s")
        wid = sid * NC + cid
        row0 = wid * cpt

        pltpu.sync_copy(src_hbm.at[pl.ds(row0, cpt)], src_v)

        zeros16 = jnp.zeros((LANES,), jnp.float32)

        @pl.loop(0, CHUNK)
        def _zero(e):
            for cc in range(CP // LANES):
                rows0[e, pl.ds(cc * LANES, LANES)] = zeros16

        slab0 = sid * slab
        for i in range(slab // CHUNK):
            pltpu.sync_copy(rows0, acc_sh.at[pl.ds(slab0 + i * CHUNK, CHUNK)])
        plsc.subcore_barrier()

        def issue(j, buf, sem):
            pltpu.async_copy(h_hbm.at[src_v.at[j]], buf, sem)

        def wait(j, buf, sem):
            pltpu.make_async_copy(h_hbm.at[src_v.at[j]], buf, sem).wait()

        def process(jl, buf):
            @pl.loop(0, CHUNK // LANES)
            def _scale(e16):
                av = att_v[jl, pl.ds(e16 * LANES, LANES)]
                for l in range(LANES):
                    a = av[l]
                    e = e16 * LANES + l
                    for cc in range(CP // LANES):
                        sl = pl.ds(cc * LANES, LANES)
                        buf[e, sl] = buf[e, sl] * a

            pltpu.sync_copy(buf, acc_sh.at[dst_v.at[jl]], add=True)

        issue(0, rows0, sem0)
        for b in range(nblk):
            pltpu.sync_copy(dst_hbm.at[pl.ds(row0 + b * BLK, BLK)], dst_v)
            pltpu.sync_copy(att_hbm.at[pl.ds(row0 + b * BLK, BLK)], att_v)

            @pl.loop(0, BLK, step=2)
            def _edges(jl):
                j = b * BLK + jl
                issue(j + 1, rows1, sem1)
                wait(j, rows0, sem0)
                process(jl, rows0)

                @pl.when(j + 2 < cpt)
                def _():
                    issue(j + 2, rows0, sem0)

                wait(j + 1, rows1, sem1)
                process(jl + 1, rows1)

        plsc.subcore_barrier()
        pltpu.sync_copy(acc_sh.at[pl.ds(slab0, slab)],
                        out_hbm.at[cid, pl.ds(slab0, slab)])

    mesh = plsc.VectorSubcoreMesh(core_axis_name="c", subcore_axis_name="s")
    return pl.kernel(
        body,
        out_type=jax.ShapeDtypeStruct((NC, NPAD, CP), jnp.float32),
        mesh=mesh,
        compiler_params=pltpu.CompilerParams(use_tc_tiling_on_sc=False),
        scratch_types=[
            pltpu.VMEM((cpt, CHUNK), jnp.int32),
            pltpu.VMEM((BLK, CHUNK), jnp.int32),
            pltpu.VMEM((BLK, CHUNK), jnp.float32),
            pltpu.VMEM((CHUNK, CP), jnp.float32),
            pltpu.VMEM((CHUNK, CP), jnp.float32),
            pltpu.VMEM_SHARED((NPAD, CP), jnp.float32),
            pltpu.SemaphoreType.DMA,
            pltpu.SemaphoreType.DMA,
        ],
    )


def kernel(x, edge_index, hop_edge_index, hop_edge_att, W1, b1, W2, b2):
    w2p = jnp.pad(W2, ((0, 0), (0, CP - C)))
    b2p = jnp.pad(b2, (0, CP - C)).reshape(1, CP)
    b1r = b1.reshape(1, H)
    h = pl.pallas_call(
        _mlp_body,
        grid=(N // RBLK,),
        in_specs=[
            pl.BlockSpec((RBLK, D), lambda i: (i, 0)),
            pl.BlockSpec((D, H), lambda i: (0, 0)),
            pl.BlockSpec((1, H), lambda i: (0, 0)),
            pl.BlockSpec((H, CP), lambda i: (0, 0)),
            pl.BlockSpec((1, CP), lambda i: (0, 0)),
        ],
        out_specs=pl.BlockSpec((RBLK, CP), lambda i: (i, 0)),
        out_shape=jax.ShapeDtypeStruct((N, CP), jnp.float32),
    )(x, W1, b1r, w2p, b2p)

    src = hop_edge_index[:, 0, :].reshape(-1)
    dst = hop_edge_index[:, 1, :].reshape(-1)
    att = hop_edge_att.reshape(-1)
    etot = att.shape[0]
    cpt = -(-etot // (NW * CHUNK))
    cpt = -(-cpt // BLK) * BLK
    epad = NW * cpt * CHUNK
    pad = epad - etot
    dst_fill = (jnp.arange(pad, dtype=dst.dtype) * 79) % N
    src = jnp.concatenate([src, jnp.zeros((pad,), src.dtype)]).reshape(-1, CHUNK)
    dst = jnp.concatenate([dst, dst_fill]).reshape(-1, CHUNK)
    att = jnp.concatenate([att, jnp.zeros((pad,), att.dtype)]).reshape(-1, CHUNK)

    acc = _make_prop(cpt)(h, src, dst, att)

    out = pl.pallas_call(
        _combine_body,
        grid=(N // RBLK,),
        in_specs=[
            pl.BlockSpec((RBLK, CP), lambda i: (i, 0)),
            pl.BlockSpec((RBLK, CP), lambda i: (i, 0)),
            pl.BlockSpec((RBLK, CP), lambda i: (i, 0)),
        ],
        out_specs=pl.BlockSpec((RBLK, CP), lambda i: (i, 0)),
        out_shape=jax.ShapeDtypeStruct((N, CP), jnp.float32),
    )(h, acc[0, :N], acc[1, :N])
    return out[:, :C]


# speedup vs baseline: 12.1967x; 2.1572x over previous
import functools

import jax
import jax.numpy as jnp
from jax import lax
from jax.experimental import pallas as pl
from jax.experimental.pallas import tpu as pltpu
from jax.experimental.pallas import tpu_sc as plsc

N = 10000
D = 128
H = 256
C = 40
CP = 48
NPAD = 10240
NC, NS, LANES = 2, 16, 16
NW = NC * NS
CHUNK = 128
RBLK = 400


def _mlp_body(x_ref, w1_ref, b1_ref, w2_ref, b2_ref, h_ref):
    h1 = jnp.dot(x_ref[...], w1_ref[...], preferred_element_type=jnp.float32)
    h1 = jnp.maximum(h1 + b1_ref[...], 0.0)
    h_ref[...] = jnp.dot(h1, w2_ref[...],
                         preferred_element_type=jnp.float32) + b2_ref[...]


def _combine_body(h_ref, a0_ref, a1_ref, o_ref):
    s = h_ref[...] + a0_ref[...] + a1_ref[...]
    col = lax.broadcasted_iota(jnp.int32, (RBLK, CP), 1)
    valid = col < C
    masked = jnp.where(valid, s, -jnp.inf)
    m = jnp.max(masked, axis=1, keepdims=True)
    ex = jnp.where(valid, jnp.exp(s - m), 0.0)
    lse = jnp.log(jnp.sum(ex, axis=1, keepdims=True)) + m
    o_ref[...] = s - lse


BLK = 80


def _make_prop(cpt):
    slab = NPAD // NS
    nblk = cpt // BLK

    def body(h_hbm, src_hbm, dst_hbm, att_hbm, out_hbm,
             src_v, dst_v, att_v, rows0, rows1, acc_sh, sem0, sem1):
        cid = lax.axis_index("c")
        sid = lax.axis_index("s")
        wid = sid * NC + cid
        row0 = wid * cpt

        pltpu.sync_copy(src_hbm.at[pl.ds(row0, cpt)], src_v)

        zeros16 = jnp.zeros((LANES,), jnp.float32)

        @pl.loop(0, CHUNK)
        def _zero(e):
            for cc in range(CP // LANES):
                rows0[e, pl.ds(cc * LANES, LANES)] = zeros16

        slab0 = sid * slab
        for i in range(slab // CHUNK):
            pltpu.sync_copy(rows0, acc_sh.at[pl.ds(slab0 + i * CHUNK, CHUNK)])
        plsc.subcore_barrier()

        def issue(j, buf, sem):
            pltpu.async_copy(h_hbm.at[src_v.at[j]], buf, sem)

        def wait(j, buf, sem):
            pltpu.make_async_copy(h_hbm.at[src_v.at[j]], buf, sem).wait()

        def process(jl, buf):
            @pl.loop(0, CHUNK // LANES)
            def _scale(e16):
                av = att_v[jl, pl.ds(e16 * LANES, LANES)]
                for l in range(LANES):
                    a = av[l]
                    e = e16 * LANES + l
                    for cc in range(CP // LANES):
                        sl = pl.ds(cc * LANES, LANES)
                        buf[e, sl] = buf[e, sl] * a

            pltpu.sync_copy(buf, acc_sh.at[dst_v.at[jl]], add=True)

        issue(0, rows0, sem0)
        for b in range(nblk):
            pltpu.sync_copy(dst_hbm.at[pl.ds(row0 + b * BLK, BLK)], dst_v)
            pltpu.sync_copy(att_hbm.at[pl.ds(row0 + b * BLK, BLK)], att_v)

            @pl.loop(0, BLK, step=2)
            def _edges(jl):
                j = b * BLK + jl
                issue(j + 1, rows1, sem1)
                wait(j, rows0, sem0)
                process(jl, rows0)

                @pl.when(j + 2 < cpt)
                def _():
                    issue(j + 2, rows0, sem0)

                wait(j + 1, rows1, sem1)
                process(jl + 1, rows1)

        plsc.subcore_barrier()
        pltpu.sync_copy(acc_sh.at[pl.ds(slab0, slab)],
                        out_hbm.at[cid, pl.ds(slab0, slab)])

    mesh = plsc.VectorSubcoreMesh(core_axis_name="c", subcore_axis_name="s")
    return pl.kernel(
        body,
        out_type=jax.ShapeDtypeStruct((NC, NPAD, CP), jnp.float32),
        mesh=mesh,
        compiler_params=pltpu.CompilerParams(use_tc_tiling_on_sc=False),
        scratch_types=[
            pltpu.VMEM((cpt, CHUNK), jnp.int32),
            pltpu.VMEM((BLK, CHUNK), jnp.int32),
            pltpu.VMEM((BLK, CHUNK), jnp.float32),
            pltpu.VMEM((CHUNK, CP), jnp.float32),
            pltpu.VMEM((CHUNK, CP), jnp.float32),
            pltpu.VMEM_SHARED((NPAD, CP), jnp.float32),
            pltpu.SemaphoreType.DMA,
            pltpu.SemaphoreType.DMA,
        ],
    )


def kernel(x, edge_index, hop_edge_index, hop_edge_att, W1, b1, W2, b2):
    w2p = jnp.pad(W2, ((0, 0), (0, CP - C)))
    b2p = jnp.pad(b2, (0, CP - C)).reshape(1, CP)
    b1r = b1.reshape(1, H)
    h = pl.pallas_call(
        _mlp_body,
        grid=(N // RBLK,),
        in_specs=[
            pl.BlockSpec((RBLK, D), lambda i: (i, 0)),
            pl.BlockSpec((D, H), lambda i: (0, 0)),
            pl.BlockSpec((1, H), lambda i: (0, 0)),
            pl.BlockSpec((H, CP), lambda i: (0, 0)),
            pl.BlockSpec((1, CP), lambda i: (0, 0)),
        ],
        out_specs=pl.BlockSpec((RBLK, CP), lambda i: (i, 0)),
        out_shape=jax.ShapeDtypeStruct((N, CP), jnp.float32),
    )(x, W1, b1r, w2p, b2p)

    src = hop_edge_index[:, 0, :].reshape(-1)
    dst = hop_edge_index[:, 1, :].reshape(-1)
    att = hop_edge_att.reshape(-1)
    etot = att.shape[0]
    cpt = -(-etot // (NW * CHUNK))
    cpt = -(-cpt // BLK) * BLK
    epad = NW * cpt * CHUNK
    pad = epad - etot
    ar = jnp.arange(pad, dtype=dst.dtype)
    src_fill = (ar * 9973) % N
    dst_fill = (ar * 9973 + 4999) % N
    src = jnp.concatenate([src, src_fill]).reshape(-1, CHUNK)
    dst = jnp.concatenate([dst, dst_fill]).reshape(-1, CHUNK)
    att = jnp.concatenate([att, jnp.zeros((pad,), att.dtype)]).reshape(-1, CHUNK)

    acc = _make_prop(cpt)(h, src, dst, att)

    out = pl.pallas_call(
        _combine_body,
        grid=(N // RBLK,),
        in_specs=[
            pl.BlockSpec((RBLK, CP), lambda i: (i, 0)),
            pl.BlockSpec((RBLK, CP), lambda i: (i, 0)),
            pl.BlockSpec((RBLK, CP), lambda i: (i, 0)),
        ],
        out_specs=pl.BlockSpec((RBLK, CP), lambda i: (i, 0)),
        out_shape=jax.ShapeDtypeStruct((N, CP), jnp.float32),
    )(h, acc[0, :N], acc[1, :N])
    return out[:, :C]


# speedup vs baseline: 13.2908x; 1.0897x over previous
import functools

import jax
import jax.numpy as jnp
from jax import lax
from jax.experimental import pallas as pl
from jax.experimental.pallas import tpu as pltpu
from jax.experimental.pallas import tpu_sc as plsc

N = 10000
D = 128
H = 256
C = 40
CP = 48
NPAD = 10240
NC, NS, LANES = 2, 16, 16
NW = NC * NS
CHUNK = 128
RBLK = 400


def _mlp_body(x_ref, w1_ref, b1_ref, w2_ref, b2_ref, h_ref):
    h1 = jnp.dot(x_ref[...], w1_ref[...], preferred_element_type=jnp.float32)
    h1 = jnp.maximum(h1 + b1_ref[...], 0.0)
    h_ref[...] = jnp.dot(h1, w2_ref[...],
                         preferred_element_type=jnp.float32) + b2_ref[...]


def _combine_body(h_ref, a0_ref, a1_ref, o_ref):
    s = h_ref[...] + a0_ref[...] + a1_ref[...]
    col = lax.broadcasted_iota(jnp.int32, (RBLK, CP), 1)
    valid = col < C
    masked = jnp.where(valid, s, -jnp.inf)
    m = jnp.max(masked, axis=1, keepdims=True)
    ex = jnp.where(valid, jnp.exp(s - m), 0.0)
    lse = jnp.log(jnp.sum(ex, axis=1, keepdims=True)) + m
    o_ref[...] = s - lse


BLK = 80
RING = 4


def _make_prop(cpt):
    slab = NPAD // NS
    nblk = cpt // BLK

    def body(h_hbm, src_hbm, dst_hbm, att_hbm, out_hbm,
             src_v, dst_v, att_v, rows, acc_sh, gsem, ssem):
        cid = lax.axis_index("c")
        sid = lax.axis_index("s")
        wid = sid * NC + cid
        row0 = wid * cpt

        pltpu.sync_copy(src_hbm.at[pl.ds(row0, cpt)], src_v)

        zeros16 = jnp.zeros((LANES,), jnp.float32)

        @pl.loop(0, CHUNK)
        def _zero(e):
            for cc in range(CP // LANES):
                rows[0][e, pl.ds(cc * LANES, LANES)] = zeros16

        slab0 = sid * slab
        for i in range(slab // CHUNK):
            pltpu.sync_copy(rows[0], acc_sh.at[pl.ds(slab0 + i * CHUNK, CHUNK)])
        plsc.subcore_barrier()

        def issue_g(j, t):
            pltpu.async_copy(h_hbm.at[src_v.at[j]], rows[t], gsem[t])

        def wait_g(j, t):
            pltpu.make_async_copy(h_hbm.at[src_v.at[j]], rows[t],
                                  gsem[t]).wait()

        def issue_s(jl, t):
            pltpu.async_copy(rows[t], acc_sh.at[dst_v.at[jl]], ssem[t],
                             add=True)

        def wait_s(jl, t):
            pltpu.make_async_copy(rows[t], acc_sh.at[dst_v.at[jl]],
                                  ssem[t]).wait()

        def scale(jl, t):
            @pl.loop(0, CHUNK // LANES)
            def _scale(e16):
                av = att_v[jl, pl.ds(e16 * LANES, LANES)]
                for l in range(LANES):
                    a = av[l]
                    e = e16 * LANES + l
                    for cc in range(CP // LANES):
                        sl = pl.ds(cc * LANES, LANES)
                        rows[t][e, sl] = rows[t][e, sl] * a

        for t in range(RING):
            issue_g(t, t)
        for b in range(nblk):
            pltpu.sync_copy(dst_hbm.at[pl.ds(row0 + b * BLK, BLK)], dst_v)
            pltpu.sync_copy(att_hbm.at[pl.ds(row0 + b * BLK, BLK)], att_v)

            @pl.loop(0, BLK, step=RING)
            def _edges(jj):
                for t in range(RING):
                    jl = jj + t
                    wait_g(b * BLK + jl, t)
                    scale(jl, t)
                    issue_s(jl, t)
                for t in range(RING):
                    jl = jj + t
                    wait_s(jl, t)
                    gn = b * BLK + jl + RING

                    @pl.when(gn < cpt)
                    def _():
                        issue_g(gn, t)

        plsc.subcore_barrier()
        pltpu.sync_copy(acc_sh.at[pl.ds(slab0, slab)],
                        out_hbm.at[cid, pl.ds(slab0, slab)])

    mesh = plsc.VectorSubcoreMesh(core_axis_name="c", subcore_axis_name="s")
    return pl.kernel(
        body,
        out_type=jax.ShapeDtypeStruct((NC, NPAD, CP), jnp.float32),
        mesh=mesh,
        compiler_params=pltpu.CompilerParams(use_tc_tiling_on_sc=False),
        scratch_types=[
            pltpu.VMEM((cpt, CHUNK), jnp.int32),
            pltpu.VMEM((BLK, CHUNK), jnp.int32),
            pltpu.VMEM((BLK, CHUNK), jnp.float32),
            [pltpu.VMEM((CHUNK, CP), jnp.float32) for _ in range(RING)],
            pltpu.VMEM_SHARED((NPAD, CP), jnp.float32),
            [pltpu.SemaphoreType.DMA for _ in range(RING)],
            [pltpu.SemaphoreType.DMA for _ in range(RING)],
        ],
    )


def kernel(x, edge_index, hop_edge_index, hop_edge_att, W1, b1, W2, b2):
    w2p = jnp.pad(W2, ((0, 0), (0, CP - C)))
    b2p = jnp.pad(b2, (0, CP - C)).reshape(1, CP)
    b1r = b1.reshape(1, H)
    h = pl.pallas_call(
        _mlp_body,
        grid=(N // RBLK,),
        in_specs=[
            pl.BlockSpec((RBLK, D), lambda i: (i, 0)),
            pl.BlockSpec((D, H), lambda i: (0, 0)),
            pl.BlockSpec((1, H), lambda i: (0, 0)),
            pl.BlockSpec((H, CP), lambda i: (0, 0)),
            pl.BlockSpec((1, CP), lambda i: (0, 0)),
        ],
        out_specs=pl.BlockSpec((RBLK, CP), lambda i: (i, 0)),
        out_shape=jax.ShapeDtypeStruct((N, CP), jnp.float32),
    )(x, W1, b1r, w2p, b2p)

    src = hop_edge_index[:, 0, :].reshape(-1)
    dst = hop_edge_index[:, 1, :].reshape(-1)
    att = hop_edge_att.reshape(-1)
    etot = att.shape[0]
    cpt = -(-etot // (NW * CHUNK))
    cpt = -(-cpt // BLK) * BLK
    epad = NW * cpt * CHUNK
    pad = epad - etot
    ar = jnp.arange(pad, dtype=dst.dtype)
    src_fill = (ar * 9973) % N
    dst_fill = (ar * 9973 + 4999) % N
    src = jnp.concatenate([src, src_fill]).reshape(-1, CHUNK)
    dst = jnp.concatenate([dst, dst_fill]).reshape(-1, CHUNK)
    att = jnp.concatenate([att, jnp.zeros((pad,), att.dtype)]).reshape(-1, CHUNK)

    acc = _make_prop(cpt)(h, src, dst, att)

    out = pl.pallas_call(
        _combine_body,
        grid=(N // RBLK,),
        in_specs=[
            pl.BlockSpec((RBLK, CP), lambda i: (i, 0)),
            pl.BlockSpec((RBLK, CP), lambda i: (i, 0)),
            pl.BlockSpec((RBLK, CP), lambda i: (i, 0)),
        ],
        out_specs=pl.BlockSpec((RBLK, CP), lambda i: (i, 0)),
        out_shape=jax.ShapeDtypeStruct((N, CP), jnp.float32),
    )(h, acc[0, :N], acc[1, :N])
    return out[:, :C]


# speedup vs baseline: 16.8874x; 1.2706x over previous
import jax
import jax.numpy as jnp
from jax import lax
from jax.experimental import pallas as pl
from jax.experimental.pallas import tpu as pltpu
from jax.experimental.pallas import tpu_sc as plsc

N = 10000
D = 128
H = 256
C = 40
CP = 48
NPAD = 10240
NC, NS, LANES = 2, 16, 16
NW = NC * NS
CHUNK = 128
RING = 4
RBLK = 1000


def _mlp_body(x_ref, w1_ref, b1_ref, w2_ref, b2_ref, h_ref):
    h1 = jnp.dot(x_ref[...], w1_ref[...], preferred_element_type=jnp.float32)
    h1 = jnp.maximum(h1 + b1_ref[...], 0.0)
    h2 = jnp.dot(h1, w2_ref[...],
                 preferred_element_type=jnp.float32) + b2_ref[...]
    h_ref[...] = jnp.concatenate(
        [h2, jnp.zeros((RBLK, CP - C), jnp.float32)], axis=1)


def _combine_body(h_ref, a0_ref, a1_ref, o_ref):
    s = h_ref[...] + a0_ref[0] + a1_ref[0]
    col = lax.broadcasted_iota(jnp.int32, (RBLK, CP), 1)
    valid = col < C
    masked = jnp.where(valid, s, -jnp.inf)
    m = jnp.max(masked, axis=1, keepdims=True)
    ex = jnp.where(valid, jnp.exp(s - m), 0.0)
    lse = jnp.log(jnp.sum(ex, axis=1, keepdims=True)) + m
    o_ref[...] = (s - lse)[:, :C]


def _make_prop(nhops, npc):
    slab = NPAD // NS
    ngrp = npc // RING
    gq, grem = divmod(ngrp, NW)
    maxc = (gq + 1) * RING

    def body(h_hbm, ei_hbm, att_hbm, out_hbm,
             src_v, dst_v, att_v, rows, acc_sh, gsem, ssem):
        cid = lax.axis_index("c")
        sid = lax.axis_index("s")
        wid = sid * NC + cid
        cnt = jnp.where(wid < grem, gq + 1, gq) * RING
        start = (wid * gq + jnp.minimum(wid, grem)) * RING
        off = maxc - cnt
        start_read = start - off

        for l in range(nhops):
            pltpu.sync_copy(ei_hbm.at[2 * l, pl.ds(start_read, maxc)],
                            src_v.at[l])

        zeros16 = jnp.zeros((LANES,), jnp.float32)

        @pl.loop(0, CHUNK)
        def _zero(e):
            for cc in range(CP // LANES):
                rows[0][e, pl.ds(cc * LANES, LANES)] = zeros16

        slab0 = sid * slab
        for i in range(slab // CHUNK):
            pltpu.sync_copy(rows[0], acc_sh.at[pl.ds(slab0 + i * CHUNK, CHUNK)])
        plsc.subcore_barrier()

        def issue_g(l, jr, t):
            pltpu.async_copy(h_hbm.at[src_v.at[l, jr]], rows[t], gsem[t])

        def wait_g(l, jr, t):
            pltpu.make_async_copy(h_hbm.at[src_v.at[l, jr]], rows[t],
                                  gsem[t]).wait()

        def issue_s(jr, t):
            pltpu.async_copy(rows[t], acc_sh.at[dst_v.at[jr]], ssem[t],
                             add=True)

        def wait_s(jr, t):
            pltpu.make_async_copy(rows[t], acc_sh.at[dst_v.at[jr]],
                                  ssem[t]).wait()

        def scale(jr, t):
            @pl.loop(0, CHUNK // LANES)
            def _scale(e16):
                av = att_v[jr, pl.ds(e16 * LANES, LANES)]
                for l in range(LANES):
                    a = av[l]
                    e = e16 * LANES + l
                    for cc in range(CP // LANES):
                        sl = pl.ds(cc * LANES, LANES)
                        rows[t][e, sl] = rows[t][e, sl] * a

        for l in range(nhops):
            pltpu.sync_copy(ei_hbm.at[2 * l + 1, pl.ds(start_read, maxc)],
                            dst_v)
            pltpu.sync_copy(att_hbm.at[l, pl.ds(start_read, maxc)], att_v)
            for t in range(RING):
                issue_g(l, off + t, t)

            @pl.loop(0, cnt, step=RING)
            def _edges(jj):
                for t in range(RING):
                    jr = off + jj + t
                    wait_g(l, jr, t)
                    scale(jr, t)
                    issue_s(jr, t)
                for t in range(RING):
                    jr = off + jj + t
                    wait_s(jr, t)
                    jn = jj + t + RING

                    @pl.when(jn < cnt)
                    def _():
                        issue_g(l, off + jn, t)

        plsc.subcore_barrier()
        pltpu.sync_copy(acc_sh.at[pl.ds(slab0, slab)],
                        out_hbm.at[cid, pl.ds(slab0, slab)])

    mesh = plsc.VectorSubcoreMesh(core_axis_name="c", subcore_axis_name="s")
    return pl.kernel(
        body,
        out_type=jax.ShapeDtypeStruct((NC, NPAD, CP), jnp.float32),
        mesh=mesh,
        compiler_params=pltpu.CompilerParams(use_tc_tiling_on_sc=False),
        scratch_types=[
            pltpu.VMEM((nhops, maxc, CHUNK), jnp.int32),
            pltpu.VMEM((maxc, CHUNK), jnp.int32),
            pltpu.VMEM((maxc, CHUNK), jnp.float32),
            [pltpu.VMEM((CHUNK, CP), jnp.float32) for _ in range(RING)],
            pltpu.VMEM_SHARED((NPAD, CP), jnp.float32),
            [pltpu.SemaphoreType.DMA for _ in range(RING)],
            [pltpu.SemaphoreType.DMA for _ in range(RING)],
        ],
    )


def kernel(x, edge_index, hop_edge_index, hop_edge_att, W1, b1, W2, b2):
    h = pl.pallas_call(
        _mlp_body,
        grid=(N // RBLK,),
        in_specs=[
            pl.BlockSpec((RBLK, D), lambda i: (i, 0)),
            pl.BlockSpec((D, H), lambda i: (0, 0)),
            pl.BlockSpec((1, H), lambda i: (0, 0)),
            pl.BlockSpec((H, C), lambda i: (0, 0)),
            pl.BlockSpec((1, C), lambda i: (0, 0)),
        ],
        out_specs=pl.BlockSpec((RBLK, CP), lambda i: (i, 0)),
        out_shape=jax.ShapeDtypeStruct((N, CP), jnp.float32),
    )(x, W1, b1.reshape(1, H), W2, b2.reshape(1, C))

    nhops, _, e_per_hop = hop_edge_index.shape
    npc = e_per_hop // CHUNK
    ei6 = hop_edge_index.reshape(2 * nhops, npc, CHUNK)
    att3 = hop_edge_att.reshape(nhops, npc, CHUNK)
    acc = _make_prop(nhops, npc)(h, ei6, att3)

    return pl.pallas_call(
        _combine_body,
        grid=(N // RBLK,),
        in_specs=[
            pl.BlockSpec((RBLK, CP), lambda i: (i, 0)),
            pl.BlockSpec((1, RBLK, CP), lambda i: (0, i, 0)),
            pl.BlockSpec((1, RBLK, CP), lambda i: (1, i, 0)),
        ],
        out_specs=pl.BlockSpec((RBLK, C), lambda i: (i, 0)),
        out_shape=jax.ShapeDtypeStruct((N, C), jnp.float32),
    )(h, acc, acc)


# speedup vs baseline: 16.9897x; 1.0061x over previous
import jax
import jax.numpy as jnp
from jax import lax
from jax.experimental import pallas as pl
from jax.experimental.pallas import tpu as pltpu
from jax.experimental.pallas import tpu_sc as plsc

N = 10000
D = 128
H = 256
C = 40
CP = 48
NPAD = 10240
NC, NS, LANES = 2, 16, 16
NW = NC * NS
CHUNK = 128
RING = 4
RBLK = 1000


def _mlp_body(x_ref, w1_ref, b1_ref, w2_ref, b2_ref, h_ref):
    h1 = jnp.dot(x_ref[...], w1_ref[...], preferred_element_type=jnp.float32)
    h1 = jnp.maximum(h1 + b1_ref[...], 0.0)
    h2 = jnp.dot(h1, w2_ref[...],
                 preferred_element_type=jnp.float32) + b2_ref[...]
    h_ref[...] = jnp.concatenate(
        [h2, jnp.zeros((RBLK, CP - C), jnp.float32)], axis=1)


def _combine_body(h_ref, a0_ref, a1_ref, o_ref):
    s = h_ref[...] + a0_ref[0] + a1_ref[0]
    col = lax.broadcasted_iota(jnp.int32, (RBLK, CP), 1)
    valid = col < C
    masked = jnp.where(valid, s, -jnp.inf)
    m = jnp.max(masked, axis=1, keepdims=True)
    ex = jnp.where(valid, jnp.exp(s - m), 0.0)
    lse = jnp.log(jnp.sum(ex, axis=1, keepdims=True)) + m
    o_ref[...] = (s - lse)[:, :C]


def _make_prop(nhops, npc):
    slab = NPAD // NS
    ngrp = npc // RING
    gq, grem = divmod(ngrp, NW)
    maxc = (gq + 1) * RING

    def body(h_hbm, ei_hbm, att_hbm, out_hbm,
             src_v, att_v, dstrow_v, rows, acc_sh, gsem, ssem, dsem):
        cid = lax.axis_index("c")
        sid = lax.axis_index("s")
        wid = sid * NC + cid
        cnt = jnp.where(wid < grem, gq + 1, gq) * RING
        start = (wid * gq + jnp.minimum(wid, grem)) * RING
        off = maxc - cnt
        read0 = (start - off) * CHUNK

        for l in range(nhops):
            pltpu.sync_copy(ei_hbm.at[l, 0, pl.ds(read0, maxc * CHUNK)],
                            src_v.at[l])

        zeros16 = jnp.zeros((LANES,), jnp.float32)

        @pl.loop(0, CHUNK)
        def _zero(e):
            for cc in range(CP // LANES):
                rows[0][e, pl.ds(cc * LANES, LANES)] = zeros16

        slab0 = sid * slab
        for i in range(slab // CHUNK):
            pltpu.sync_copy(rows[0], acc_sh.at[pl.ds(slab0 + i * CHUNK, CHUNK)])
        plsc.subcore_barrier()

        def issue_g(l, jr, t):
            pltpu.async_copy(h_hbm.at[src_v.at[l, pl.ds(jr * CHUNK, CHUNK)]],
                             rows[t], gsem[t])

        def wait_g(l, jr, t):
            pltpu.make_async_copy(
                h_hbm.at[src_v.at[l, pl.ds(jr * CHUNK, CHUNK)]], rows[t],
                gsem[t]).wait()

        def issue_d(l, jr, t):
            pltpu.async_copy(ei_hbm.at[l, 1, pl.ds(read0 + jr * CHUNK, CHUNK)],
                             dstrow_v.at[t], dsem[t])

        def wait_d(l, jr, t):
            pltpu.make_async_copy(
                ei_hbm.at[l, 1, pl.ds(read0 + jr * CHUNK, CHUNK)],
                dstrow_v.at[t], dsem[t]).wait()

        def issue_s(t):
            pltpu.async_copy(rows[t], acc_sh.at[dstrow_v.at[t]], ssem[t],
                             add=True)

        def wait_s(t):
            pltpu.make_async_copy(rows[t], acc_sh.at[dstrow_v.at[t]],
                                  ssem[t]).wait()

        def scale(jr, t):
            @pl.loop(0, CHUNK // LANES)
            def _scale(e16):
                av = att_v[pl.ds(jr * CHUNK + e16 * LANES, LANES)]
                for l in range(LANES):
                    a = av[l]
                    e = e16 * LANES + l
                    for cc in range(CP // LANES):
                        sl = pl.ds(cc * LANES, LANES)
                        rows[t][e, sl] = rows[t][e, sl] * a

        for l in range(nhops):
            pltpu.sync_copy(att_hbm.at[l, pl.ds(read0, maxc * CHUNK)], att_v)
            for t in range(RING):
                issue_g(l, off + t, t)
                issue_d(l, off + t, t)

            @pl.loop(0, cnt, step=RING)
            def _edges(jj):
                for t in range(RING):
                    jr = off + jj + t
                    wait_g(l, jr, t)
                    scale(jr, t)
                    wait_d(l, jr, t)
                    issue_s(t)
                for t in range(RING):
                    jr = off + jj + t
                    wait_s(t)
                    jn = jj + t + RING

                    @pl.when(jn < cnt)
                    def _():
                        issue_g(l, off + jn, t)
                        issue_d(l, off + jn, t)

        plsc.subcore_barrier()
        pltpu.sync_copy(acc_sh.at[pl.ds(slab0, slab)],
                        out_hbm.at[cid, pl.ds(slab0, slab)])

    mesh = plsc.VectorSubcoreMesh(core_axis_name="c", subcore_axis_name="s")
    return pl.kernel(
        body,
        out_type=jax.ShapeDtypeStruct((NC, NPAD, CP), jnp.float32),
        mesh=mesh,
        compiler_params=pltpu.CompilerParams(use_tc_tiling_on_sc=False),
        scratch_types=[
            pltpu.VMEM((nhops, maxc * CHUNK), jnp.int32),
            pltpu.VMEM((maxc * CHUNK,), jnp.float32),
            pltpu.VMEM((RING, CHUNK), jnp.int32),
            [pltpu.VMEM((CHUNK, CP), jnp.float32) for _ in range(RING)],
            pltpu.VMEM_SHARED((NPAD, CP), jnp.float32),
            [pltpu.SemaphoreType.DMA for _ in range(RING)],
            [pltpu.SemaphoreType.DMA for _ in range(RING)],
            [pltpu.SemaphoreType.DMA for _ in range(RING)],
        ],
    )


def kernel(x, edge_index, hop_edge_index, hop_edge_att, W1, b1, W2, b2):
    h = pl.pallas_call(
        _mlp_body,
        grid=(N // RBLK,),
        in_specs=[
            pl.BlockSpec((RBLK, D), lambda i: (i, 0)),
            pl.BlockSpec((D, H), lambda i: (0, 0)),
            pl.BlockSpec((1, H), lambda i: (0, 0)),
            pl.BlockSpec((H, C), lambda i: (0, 0)),
            pl.BlockSpec((1, C), lambda i: (0, 0)),
        ],
        out_specs=pl.BlockSpec((RBLK, CP), lambda i: (i, 0)),
        out_shape=jax.ShapeDtypeStruct((N, CP), jnp.float32),
    )(x, W1, b1.reshape(1, H), W2, b2.reshape(1, C))

    nhops, _, e_per_hop = hop_edge_index.shape
    npc = e_per_hop // CHUNK
    acc = _make_prop(nhops, npc)(h, hop_edge_index, hop_edge_att)

    return pl.pallas_call(
        _combine_body,
        grid=(N // RBLK,),
        in_specs=[
            pl.BlockSpec((RBLK, CP), lambda i: (i, 0)),
            pl.BlockSpec((1, RBLK, CP), lambda i: (0, i, 0)),
            pl.BlockSpec((1, RBLK, CP), lambda i: (1, i, 0)),
        ],
        out_specs=pl.BlockSpec((RBLK, C), lambda i: (i, 0)),
        out_shape=jax.ShapeDtypeStruct((N, C), jnp.float32),
    )(h, acc, acc)
